# Initial kernel scaffold; baseline (speedup 1.0000x reference)
import jax
import jax.numpy as jnp
from jax.experimental import pallas as pl


def kernel(x, edge_index, edge_attr, batch, proj_W, proj_b, edge_W, edge_b, conv_W1, conv_b1, conv_W2, conv_b2, bn_gamma, bn_beta, vn_W1, vn_b1, vn_W2, vn_b2):
    raise NotImplementedError("write your pallas kernel here")


# speedup vs baseline: 3.1538x; 3.1538x over previous
import functools

import jax
import jax.numpy as jnp
import numpy as np
from jax import lax
from jax.experimental import pallas as pl
from jax.experimental.pallas import tpu as pltpu
from jax.experimental.pallas import tpu_sc as plsc

N = 10000
E = 320000
H = 128
G = 64

NC = 2
NS = 16
EPW = E // (NC * NS)
C = 80
NCHUNK = EPW // C
NPAD = 10240
RPT = NPAD // NS
ZR = 128

BN = 2000
BE = 4000

_BN_SCALE = np.float32(1.0 / np.sqrt(1.0 + 1e-5))


def _sc_aggr_body(h_hbm, e_hbm, src_hbm, dst_hbm, out_hbm,
                  src_v, dst_v, rows_v, e_v, zbuf, accum, sem):
    c = lax.axis_index("c")
    s = lax.axis_index("s")

    zero16 = jnp.zeros((16,), jnp.float32)

    def zrow(i, carry):
        for r in range(8):
            zbuf[i, pl.ds(r * 16, 16)] = zero16
        return carry

    lax.fori_loop(0, ZR, zrow, 0)
    for k in range(RPT // ZR):
        pltpu.sync_copy(zbuf, accum.at[pl.ds(s * RPT + k * ZR, ZR)])
    plsc.subcore_barrier()

    wbase = c * (NS * EPW) + s * EPW

    def chunk(j, carry):
        base = wbase + j * C
        pltpu.sync_copy(src_hbm.at[pl.ds(base, C)], src_v)
        pltpu.sync_copy(dst_hbm.at[pl.ds(base, C)], dst_v)
        cp = pltpu.async_copy(h_hbm.at[src_v], rows_v, sem)
        pltpu.sync_copy(e_hbm.at[pl.ds(base, C)], e_v)
        cp.wait()

        def msg(i, cc):
            for r in range(8):
                sl = pl.ds(r * 16, 16)
                rows_v[i, sl] = jnp.maximum(rows_v[i, sl] + e_v[i, sl], 0.0)
            return cc

        lax.fori_loop(0, C, msg, 0)
        pltpu.sync_copy(rows_v, accum.at[dst_v], add=True)
        return carry

    lax.fori_loop(0, NCHUNK, chunk, 0)
    plsc.subcore_barrier()

    for k in range(RPT // ZR):
        off = s * RPT + k * ZR
        pltpu.sync_copy(accum.at[pl.ds(off, ZR)], zbuf)
        pltpu.sync_copy(zbuf, out_hbm.at[pl.ds(c * NPAD + off, ZR)])


@functools.cache
def _sc_aggr_kernel():
    return pl.kernel(
        _sc_aggr_body,
        out_type=jax.ShapeDtypeStruct((NC * NPAD, H), jnp.float32),
        mesh=plsc.VectorSubcoreMesh(core_axis_name="c", subcore_axis_name="s",
                                    num_cores=NC, num_subcores=NS),
        scratch_types=[
            pltpu.VMEM((C,), jnp.int32),
            pltpu.VMEM((C,), jnp.int32),
            pltpu.VMEM((C, H), jnp.float32),
            pltpu.VMEM((C, H), jnp.float32),
            pltpu.VMEM((ZR, H), jnp.float32),
            pltpu.VMEM_SHARED((NPAD, H), jnp.float32),
            pltpu.SemaphoreType.DMA,
        ],
    )


def _sc_aggr(h, e, src, dst):
    return _sc_aggr_kernel()(h, e, src, dst)


def _proj_body(x_ref, w_ref, b_ref, o_ref):
    o_ref[...] = (
        jnp.dot(x_ref[...], w_ref[...], preferred_element_type=jnp.float32)
        + b_ref[...]
    )


def _edge_feat_body(a_ref, w_ref, b_ref, o_ref):
    a = a_ref[...]
    w = w_ref[...]
    acc = b_ref[...] + a[:, 0:1] * w[0:1, :]
    for j in range(1, 4):
        acc = acc + a[:, j : j + 1] * w[j : j + 1, :]
    o_ref[...] = acc


def _layer_body(h_ref, a0_ref, a1_ref, batch_ref, w1_ref, b1_ref, w2_ref,
                b2_ref, g_ref, bt_ref, ho_ref, s_ref, cnt_ref, emb_ref):
    i = pl.program_id(0)
    z = h_ref[...] + a0_ref[...] + a1_ref[...]
    t = jnp.maximum(
        jnp.dot(z, w1_ref[...], preferred_element_type=jnp.float32)
        + b1_ref[...], 0.0)
    t = jnp.dot(t, w2_ref[...], preferred_element_type=jnp.float32) + b2_ref[...]
    t = t * (g_ref[...] * _BN_SCALE) + bt_ref[...]
    ho = jnp.maximum(t, 0.0)
    ho_ref[...] = ho

    onehot = (batch_ref[...] ==
              lax.broadcasted_iota(jnp.int32, (BN, G), 1)).astype(jnp.float32)
    s_blk = lax.dot_general(onehot, ho, (((0,), (0,)), ((), ())),
                            preferred_element_type=jnp.float32)
    c_blk = lax.dot_general(onehot, jnp.ones((BN, H), jnp.float32),
                            (((0,), (0,)), ((), ())),
                            preferred_element_type=jnp.float32)

    @pl.when(i == 0)
    def _():
        s_ref[...] = jnp.zeros_like(s_ref)
        cnt_ref[...] = jnp.zeros_like(cnt_ref)

    s_ref[...] += s_blk
    cnt_ref[...] += c_blk

    @pl.when(i == pl.num_programs(0) - 1)
    def _():
        emb_ref[...] = s_ref[...] / jnp.maximum(cnt_ref[...], 1.0)


def _addvn_body(h_ref, batch_ref, s_ref, vnp_ref, w1_ref, b1_ref, w2_ref,
                b2_ref, ho_ref, vn_ref):
    u = jnp.maximum(
        jnp.dot(s_ref[...], w1_ref[...], preferred_element_type=jnp.float32)
        + b1_ref[...], 0.0)
    u = jnp.dot(u, w2_ref[...], preferred_element_type=jnp.float32) + b2_ref[...]
    vn_new = vnp_ref[...] + u
    vn_ref[...] = vn_new
    onehot = (batch_ref[...] ==
              lax.broadcasted_iota(jnp.int32, (BN, G), 1)).astype(jnp.float32)
    ho_ref[...] = h_ref[...] + jnp.dot(
        onehot, vn_new, preferred_element_type=jnp.float32)


def _row_spec(blk):
    return pl.BlockSpec((blk, H), lambda i: (i, 0))


def _full_spec(r):
    return pl.BlockSpec((r, H), lambda i: (0, 0))


_proj = pl.pallas_call(
    _proj_body,
    grid=(N // BN,),
    in_specs=[_row_spec(BN), _full_spec(H), pl.BlockSpec((1, H), lambda i: (0, 0))],
    out_specs=_row_spec(BN),
    out_shape=jax.ShapeDtypeStruct((N, H), jnp.float32),
)

_edge_feat = pl.pallas_call(
    _edge_feat_body,
    grid=(E // BE,),
    in_specs=[pl.BlockSpec((BE, 4), lambda i: (i, 0)),
              pl.BlockSpec((4, H), lambda i: (0, 0)),
              pl.BlockSpec((1, H), lambda i: (0, 0))],
    out_specs=_row_spec(BE),
    out_shape=jax.ShapeDtypeStruct((E, H), jnp.float32),
)

_layer = pl.pallas_call(
    _layer_body,
    grid=(N // BN,),
    in_specs=[_row_spec(BN), _row_spec(BN), _row_spec(BN),
              pl.BlockSpec((BN, 1), lambda i: (i, 0)),
              _full_spec(H), pl.BlockSpec((1, H), lambda i: (0, 0)),
              _full_spec(H), pl.BlockSpec((1, H), lambda i: (0, 0)),
              pl.BlockSpec((1, H), lambda i: (0, 0)),
              pl.BlockSpec((1, H), lambda i: (0, 0))],
    out_specs=[_row_spec(BN), _full_spec(G), _full_spec(G), _full_spec(G)],
    out_shape=[jax.ShapeDtypeStruct((N, H), jnp.float32),
               jax.ShapeDtypeStruct((G, H), jnp.float32),
               jax.ShapeDtypeStruct((G, H), jnp.float32),
               jax.ShapeDtypeStruct((G, H), jnp.float32)],
)

_addvn = pl.pallas_call(
    _addvn_body,
    grid=(N // BN,),
    in_specs=[_row_spec(BN),
              pl.BlockSpec((BN, 1), lambda i: (i, 0)),
              _full_spec(G), _full_spec(G),
              _full_spec(H), pl.BlockSpec((1, H), lambda i: (0, 0)),
              _full_spec(H), pl.BlockSpec((1, H), lambda i: (0, 0))],
    out_specs=[_row_spec(BN), _full_spec(G)],
    out_shape=[jax.ShapeDtypeStruct((N, H), jnp.float32),
               jax.ShapeDtypeStruct((G, H), jnp.float32)],
)


def kernel(x, edge_index, edge_attr, batch, proj_W, proj_b, edge_W, edge_b,
           conv_W1, conv_b1, conv_W2, conv_b2, bn_gamma, bn_beta,
           vn_W1, vn_b1, vn_W2, vn_b2):
    src = edge_index[0]
    dst = edge_index[1]
    batch2 = batch.reshape(N, 1)

    proj_b2 = proj_b.reshape(1, H)
    edge_b2 = edge_b.reshape(1, H)
    vn_b1_2 = vn_b1.reshape(1, H)
    vn_b2_2 = vn_b2.reshape(1, H)

    h = _proj(x, proj_W, proj_b2)
    e = _edge_feat(edge_attr, edge_W, edge_b2)

    vn = jnp.zeros((G, H), jnp.float32)
    s_prev = None
    emb = None
    for i in range(3):
        if i > 0:
            h, vn = _addvn(h, batch2, s_prev, vn, vn_W1, vn_b1_2,
                           vn_W2, vn_b2_2)
        agg = _sc_aggr(h, e, src, dst)
        h, s_prev, _cnt, emb = _layer(
            h, agg[:N], agg[NPAD:NPAD + N], batch2,
            conv_W1[i], conv_b1[i].reshape(1, H),
            conv_W2[i], conv_b2[i].reshape(1, H),
            bn_gamma[i].reshape(1, H), bn_beta[i].reshape(1, H))
    return (h, emb)


# speedup vs baseline: 5.1327x; 1.6274x over previous
import functools

import jax
import jax.numpy as jnp
import numpy as np
from jax import lax
from jax.experimental import pallas as pl
from jax.experimental.pallas import tpu as pltpu
from jax.experimental.pallas import tpu_sc as plsc

N = 10000
E = 320000
H = 128
G = 64

NC = 2
NS = 16
EPW = E // (NC * NS)
C = 80
NCHUNK = EPW // C
NPAD = 10240
RPT = NPAD // NS
ZR = 128

BN = 2000
BE = 4000

_BN_SCALE = np.float32(1.0 / np.sqrt(1.0 + 1e-5))


def _sc_aggr_body(h_hbm, e_hbm, src_hbm, dst_hbm, out_hbm,
                  srcA, dstA, srcB, dstB, rowsA, rowsB, eA, eB, accum,
                  isemA, isemB, esemA, esemB, gsemA, gsemB):
    c = lax.axis_index("c")
    s = lax.axis_index("s")
    wbase = c * (NS * EPW) + s * EPW

    banks = ((srcA, dstA, rowsA, eA, isemA, esemA, gsemA),
             (srcB, dstB, rowsB, eB, isemB, esemB, gsemB))

    def idx_copies(j, bank):
        base = wbase + j * C
        return (pltpu.make_async_copy(src_hbm.at[pl.ds(base, C)], bank[0],
                                      bank[4]),
                pltpu.make_async_copy(dst_hbm.at[pl.ds(base, C)], bank[1],
                                      bank[4]))

    def e_copy(j, bank):
        base = wbase + j * C
        return pltpu.make_async_copy(e_hbm.at[pl.ds(base, C)], bank[3],
                                     bank[5])

    def g_copy(bank):
        return pltpu.make_async_copy(h_hbm.at[bank[0]], bank[2], bank[6])

    for cp in idx_copies(0, banks[0]):
        cp.start()
    for cp in idx_copies(1, banks[1]):
        cp.start()
    for cp in idx_copies(0, banks[0]):
        cp.wait()
    g_copy(banks[0]).start()
    e_copy(0, banks[0]).start()

    zero16 = jnp.zeros((16,), jnp.float32)

    @plsc.parallel_loop(0, C, unroll=2)
    def _(i):
        for r in range(8):
            rowsB[i, pl.ds(r * 16, 16)] = zero16

    for k in range(RPT // C):
        pltpu.sync_copy(rowsB, accum.at[pl.ds(s * RPT + k * C, C)])
    plsc.subcore_barrier()

    def compute(bank):
        rows, ebuf = bank[2], bank[3]

        @plsc.parallel_loop(0, C, unroll=4)
        def _(i):
            for r in range(8):
                sl = pl.ds(r * 16, 16)
                rows[i, sl] = jnp.maximum(rows[i, sl] + ebuf[i, sl], 0.0)

    def phase(jcur, cur, nxt):
        jnext = jcur + 1
        g_copy(cur).wait()
        for cp in idx_copies(jnext, nxt):
            cp.wait()
        g_copy(nxt).start()
        e_copy(jnext, nxt).start()
        e_copy(jcur, cur).wait()
        compute(cur)
        pltpu.sync_copy(cur[2], accum.at[cur[1]], add=True)

        @pl.when(jcur + 2 < NCHUNK)
        def _():
            for cp in idx_copies(jcur + 2, cur):
                cp.start()

    def pair(j2, carry):
        phase(2 * j2, banks[0], banks[1])
        phase(2 * j2 + 1, banks[1], banks[0])
        return carry

    lax.fori_loop(0, (NCHUNK - 1) // 2, pair, 0)

    g_copy(banks[0]).wait()
    e_copy(NCHUNK - 1, banks[0]).wait()
    compute(banks[0])
    pltpu.sync_copy(banks[0][2], accum.at[banks[0][1]], add=True)

    plsc.subcore_barrier()

    nchunks_out = RPT // C

    def out_cp(k, bank):
        off = s * RPT + k * C
        return pltpu.make_async_copy(bank[2],
                                     out_hbm.at[pl.ds(c * NPAD + off, C)],
                                     bank[6])

    for k in range(nchunks_out):
        bank = banks[k % 2]
        if k >= 2:
            out_cp(k - 2, bank).wait()
        pltpu.sync_copy(accum.at[pl.ds(s * RPT + k * C, C)], bank[2])
        out_cp(k, bank).start()
    out_cp(nchunks_out - 2, banks[nchunks_out % 2]).wait()
    out_cp(nchunks_out - 1, banks[(nchunks_out - 1) % 2]).wait()


@functools.cache
def _sc_aggr_kernel():
    return pl.kernel(
        _sc_aggr_body,
        out_type=jax.ShapeDtypeStruct((NC * NPAD, H), jnp.float32),
        mesh=plsc.VectorSubcoreMesh(core_axis_name="c", subcore_axis_name="s",
                                    num_cores=NC, num_subcores=NS),
        scratch_types=[
            pltpu.VMEM((C,), jnp.int32),
            pltpu.VMEM((C,), jnp.int32),
            pltpu.VMEM((C,), jnp.int32),
            pltpu.VMEM((C,), jnp.int32),
            pltpu.VMEM((C, H), jnp.float32),
            pltpu.VMEM((C, H), jnp.float32),
            pltpu.VMEM((C, H), jnp.float32),
            pltpu.VMEM((C, H), jnp.float32),
            pltpu.VMEM_SHARED((NPAD, H), jnp.float32),
            pltpu.SemaphoreType.DMA,
            pltpu.SemaphoreType.DMA,
            pltpu.SemaphoreType.DMA,
            pltpu.SemaphoreType.DMA,
            pltpu.SemaphoreType.DMA,
            pltpu.SemaphoreType.DMA,
        ],
    )


def _sc_aggr(h, e, src, dst):
    return _sc_aggr_kernel()(h, e, src, dst)


def _proj_body(x_ref, w_ref, b_ref, o_ref):
    o_ref[...] = (
        jnp.dot(x_ref[...], w_ref[...], preferred_element_type=jnp.float32)
        + b_ref[...]
    )


def _edge_feat_body(a_ref, w_ref, b_ref, o_ref):
    a = a_ref[...]
    w = w_ref[...]
    acc = b_ref[...] + a[:, 0:1] * w[0:1, :]
    for j in range(1, 4):
        acc = acc + a[:, j : j + 1] * w[j : j + 1, :]
    o_ref[...] = acc


def _layer_body(h_ref, a0_ref, a1_ref, batch_ref, w1_ref, b1_ref, w2_ref,
                b2_ref, g_ref, bt_ref, ho_ref, s_ref, cnt_ref, emb_ref):
    i = pl.program_id(0)
    z = h_ref[...] + a0_ref[...] + a1_ref[...]
    t = jnp.maximum(
        jnp.dot(z, w1_ref[...], preferred_element_type=jnp.float32)
        + b1_ref[...], 0.0)
    t = jnp.dot(t, w2_ref[...], preferred_element_type=jnp.float32) + b2_ref[...]
    t = t * (g_ref[...] * _BN_SCALE) + bt_ref[...]
    ho = jnp.maximum(t, 0.0)
    ho_ref[...] = ho

    onehot = (batch_ref[...] ==
              lax.broadcasted_iota(jnp.int32, (BN, G), 1)).astype(jnp.float32)
    s_blk = lax.dot_general(onehot, ho, (((0,), (0,)), ((), ())),
                            preferred_element_type=jnp.float32)
    c_blk = lax.dot_general(onehot, jnp.ones((BN, H), jnp.float32),
                            (((0,), (0,)), ((), ())),
                            preferred_element_type=jnp.float32)

    @pl.when(i == 0)
    def _():
        s_ref[...] = jnp.zeros_like(s_ref)
        cnt_ref[...] = jnp.zeros_like(cnt_ref)

    s_ref[...] += s_blk
    cnt_ref[...] += c_blk

    @pl.when(i == pl.num_programs(0) - 1)
    def _():
        emb_ref[...] = s_ref[...] / jnp.maximum(cnt_ref[...], 1.0)


def _addvn_body(h_ref, batch_ref, s_ref, vnp_ref, w1_ref, b1_ref, w2_ref,
                b2_ref, ho_ref, vn_ref):
    u = jnp.maximum(
        jnp.dot(s_ref[...], w1_ref[...], preferred_element_type=jnp.float32)
        + b1_ref[...], 0.0)
    u = jnp.dot(u, w2_ref[...], preferred_element_type=jnp.float32) + b2_ref[...]
    vn_new = vnp_ref[...] + u
    vn_ref[...] = vn_new
    onehot = (batch_ref[...] ==
              lax.broadcasted_iota(jnp.int32, (BN, G), 1)).astype(jnp.float32)
    ho_ref[...] = h_ref[...] + jnp.dot(
        onehot, vn_new, preferred_element_type=jnp.float32)


def _row_spec(blk):
    return pl.BlockSpec((blk, H), lambda i: (i, 0))


def _full_spec(r):
    return pl.BlockSpec((r, H), lambda i: (0, 0))


_proj = pl.pallas_call(
    _proj_body,
    grid=(N // BN,),
    in_specs=[_row_spec(BN), _full_spec(H), pl.BlockSpec((1, H), lambda i: (0, 0))],
    out_specs=_row_spec(BN),
    out_shape=jax.ShapeDtypeStruct((N, H), jnp.float32),
)

_edge_feat = pl.pallas_call(
    _edge_feat_body,
    grid=(E // BE,),
    in_specs=[pl.BlockSpec((BE, 4), lambda i: (i, 0)),
              pl.BlockSpec((4, H), lambda i: (0, 0)),
              pl.BlockSpec((1, H), lambda i: (0, 0))],
    out_specs=_row_spec(BE),
    out_shape=jax.ShapeDtypeStruct((E, H), jnp.float32),
)

_layer = pl.pallas_call(
    _layer_body,
    grid=(N // BN,),
    in_specs=[_row_spec(BN), _row_spec(BN), _row_spec(BN),
              pl.BlockSpec((BN, 1), lambda i: (i, 0)),
              _full_spec(H), pl.BlockSpec((1, H), lambda i: (0, 0)),
              _full_spec(H), pl.BlockSpec((1, H), lambda i: (0, 0)),
              pl.BlockSpec((1, H), lambda i: (0, 0)),
              pl.BlockSpec((1, H), lambda i: (0, 0))],
    out_specs=[_row_spec(BN), _full_spec(G), _full_spec(G), _full_spec(G)],
    out_shape=[jax.ShapeDtypeStruct((N, H), jnp.float32),
               jax.ShapeDtypeStruct((G, H), jnp.float32),
               jax.ShapeDtypeStruct((G, H), jnp.float32),
               jax.ShapeDtypeStruct((G, H), jnp.float32)],
)

_addvn = pl.pallas_call(
    _addvn_body,
    grid=(N // BN,),
    in_specs=[_row_spec(BN),
              pl.BlockSpec((BN, 1), lambda i: (i, 0)),
              _full_spec(G), _full_spec(G),
              _full_spec(H), pl.BlockSpec((1, H), lambda i: (0, 0)),
              _full_spec(H), pl.BlockSpec((1, H), lambda i: (0, 0))],
    out_specs=[_row_spec(BN), _full_spec(G)],
    out_shape=[jax.ShapeDtypeStruct((N, H), jnp.float32),
               jax.ShapeDtypeStruct((G, H), jnp.float32)],
)


def kernel(x, edge_index, edge_attr, batch, proj_W, proj_b, edge_W, edge_b,
           conv_W1, conv_b1, conv_W2, conv_b2, bn_gamma, bn_beta,
           vn_W1, vn_b1, vn_W2, vn_b2):
    src = edge_index[0]
    dst = edge_index[1]
    batch2 = batch.reshape(N, 1)

    proj_b2 = proj_b.reshape(1, H)
    edge_b2 = edge_b.reshape(1, H)
    vn_b1_2 = vn_b1.reshape(1, H)
    vn_b2_2 = vn_b2.reshape(1, H)

    h = _proj(x, proj_W, proj_b2)
    e = _edge_feat(edge_attr, edge_W, edge_b2)

    vn = jnp.zeros((G, H), jnp.float32)
    s_prev = None
    emb = None
    for i in range(3):
        if i > 0:
            h, vn = _addvn(h, batch2, s_prev, vn, vn_W1, vn_b1_2,
                           vn_W2, vn_b2_2)
        agg = _sc_aggr(h, e, src, dst)
        h, s_prev, _cnt, emb = _layer(
            h, agg[:N], agg[NPAD:NPAD + N], batch2,
            conv_W1[i], conv_b1[i].reshape(1, H),
            conv_W2[i], conv_b2[i].reshape(1, H),
            bn_gamma[i].reshape(1, H), bn_beta[i].reshape(1, H))
    return (h, emb)


# speedup vs baseline: 5.3447x; 1.0413x over previous
import functools

import jax
import jax.numpy as jnp
import numpy as np
from jax import lax
from jax.experimental import pallas as pl
from jax.experimental.pallas import tpu as pltpu
from jax.experimental.pallas import tpu_sc as plsc

N = 10000
E = 320000
H = 128
G = 64

NC = 2
NS = 16
EPW = E // (NC * NS)
C = 80
NCHUNK = EPW // C
NPAD = 10240
RPT = NPAD // NS
ZR = 128

BN = 2000
BE = 4000

_BN_SCALE = np.float32(1.0 / np.sqrt(1.0 + 1e-5))


def _sc_aggr_body(h_hbm, e_hbm, src_hbm, dst_hbm, out_hbm,
                  srcA, dstA, srcB, dstB, rowsA, rowsB, eA, eB, accum,
                  isemA, isemB, esemA, esemB, gsemA, gsemB, ssemA, ssemB):
    c = lax.axis_index("c")
    s = lax.axis_index("s")
    wbase = c * (NS * EPW) + s * EPW

    banks = ((srcA, dstA, rowsA, eA, isemA, esemA, gsemA, ssemA),
             (srcB, dstB, rowsB, eB, isemB, esemB, gsemB, ssemB))

    def scat_start(bank):
        pltpu.async_copy(bank[2], accum.at[bank[1]], bank[7], add=True)

    def scat_wait(bank):
        pltpu.make_async_copy(bank[2], accum.at[bank[1]], bank[7]).wait()

    def idx_copies(j, bank):
        base = wbase + j * C
        return (pltpu.make_async_copy(src_hbm.at[pl.ds(base, C)], bank[0],
                                      bank[4]),
                pltpu.make_async_copy(dst_hbm.at[pl.ds(base, C)], bank[1],
                                      bank[4]))

    def e_copy(j, bank):
        base = wbase + j * C
        return pltpu.make_async_copy(e_hbm.at[pl.ds(base, C)], bank[3],
                                     bank[5])

    def g_copy(bank):
        return pltpu.make_async_copy(h_hbm.at[bank[0]], bank[2], bank[6])

    for cp in idx_copies(0, banks[0]):
        cp.start()
    for cp in idx_copies(1, banks[1]):
        cp.start()
    for cp in idx_copies(0, banks[0]):
        cp.wait()
    g_copy(banks[0]).start()
    e_copy(0, banks[0]).start()

    zero16 = jnp.zeros((16,), jnp.float32)

    @plsc.parallel_loop(0, C, unroll=2)
    def _(i):
        for r in range(8):
            rowsB[i, pl.ds(r * 16, 16)] = zero16

    for k in range(RPT // C):
        pltpu.sync_copy(rowsB, accum.at[pl.ds(s * RPT + k * C, C)])
    plsc.subcore_barrier()

    pltpu.async_copy(rowsB, accum.at[dstA], ssemB, add=True)

    def compute(bank):
        rows, ebuf = bank[2], bank[3]

        @plsc.parallel_loop(0, C, unroll=4)
        def _(i):
            for r in range(8):
                sl = pl.ds(r * 16, 16)
                rows[i, sl] = jnp.maximum(rows[i, sl] + ebuf[i, sl], 0.0)

    def phase(jcur, cur, nxt):
        jnext = jcur + 1
        g_copy(cur).wait()
        for cp in idx_copies(jnext, nxt):
            cp.wait()
        scat_wait(nxt)
        g_copy(nxt).start()
        e_copy(jnext, nxt).start()
        e_copy(jcur, cur).wait()
        compute(cur)
        scat_start(cur)

        @pl.when(jcur + 2 < NCHUNK)
        def _():
            for cp in idx_copies(jcur + 2, cur):
                cp.start()

    def pair(j2, carry):
        phase(2 * j2, banks[0], banks[1])
        phase(2 * j2 + 1, banks[1], banks[0])
        return carry

    lax.fori_loop(0, (NCHUNK - 1) // 2, pair, 0)

    g_copy(banks[0]).wait()
    e_copy(NCHUNK - 1, banks[0]).wait()
    compute(banks[0])
    scat_start(banks[0])
    scat_wait(banks[1])
    scat_wait(banks[0])

    plsc.subcore_barrier()

    nchunks_out = RPT // C

    def out_cp(k, bank):
        off = s * RPT + k * C
        return pltpu.make_async_copy(bank[2],
                                     out_hbm.at[pl.ds(c * NPAD + off, C)],
                                     bank[6])

    for k in range(nchunks_out):
        bank = banks[k % 2]
        if k >= 2:
            out_cp(k - 2, bank).wait()
        pltpu.sync_copy(accum.at[pl.ds(s * RPT + k * C, C)], bank[2])
        out_cp(k, bank).start()
    out_cp(nchunks_out - 2, banks[nchunks_out % 2]).wait()
    out_cp(nchunks_out - 1, banks[(nchunks_out - 1) % 2]).wait()


@functools.cache
def _sc_aggr_kernel():
    return pl.kernel(
        _sc_aggr_body,
        out_type=jax.ShapeDtypeStruct((NC * NPAD, H), jnp.float32),
        mesh=plsc.VectorSubcoreMesh(core_axis_name="c", subcore_axis_name="s",
                                    num_cores=NC, num_subcores=NS),
        scratch_types=[
            pltpu.VMEM((C,), jnp.int32),
            pltpu.VMEM((C,), jnp.int32),
            pltpu.VMEM((C,), jnp.int32),
            pltpu.VMEM((C,), jnp.int32),
            pltpu.VMEM((C, H), jnp.float32),
            pltpu.VMEM((C, H), jnp.float32),
            pltpu.VMEM((C, H), jnp.float32),
            pltpu.VMEM((C, H), jnp.float32),
            pltpu.VMEM_SHARED((NPAD, H), jnp.float32),
            pltpu.SemaphoreType.DMA,
            pltpu.SemaphoreType.DMA,
            pltpu.SemaphoreType.DMA,
            pltpu.SemaphoreType.DMA,
            pltpu.SemaphoreType.DMA,
            pltpu.SemaphoreType.DMA,
            pltpu.SemaphoreType.DMA,
            pltpu.SemaphoreType.DMA,
        ],
    )


def _sc_aggr(h, e, src, dst):
    return _sc_aggr_kernel()(h, e, src, dst)


def _proj_body(x_ref, w_ref, b_ref, o_ref):
    o_ref[...] = (
        jnp.dot(x_ref[...], w_ref[...], preferred_element_type=jnp.float32)
        + b_ref[...]
    )


def _edge_feat_body(a_ref, w_ref, b_ref, o_ref):
    a = a_ref[...]
    w = w_ref[...]
    acc = b_ref[...] + a[:, 0:1] * w[0:1, :]
    for j in range(1, 4):
        acc = acc + a[:, j : j + 1] * w[j : j + 1, :]
    o_ref[...] = acc


def _layer_body(h_ref, a0_ref, a1_ref, batch_ref, w1_ref, b1_ref, w2_ref,
                b2_ref, g_ref, bt_ref, ho_ref, s_ref, cnt_ref, emb_ref):
    i = pl.program_id(0)
    z = h_ref[...] + a0_ref[...] + a1_ref[...]
    t = jnp.maximum(
        jnp.dot(z, w1_ref[...], preferred_element_type=jnp.float32)
        + b1_ref[...], 0.0)
    t = jnp.dot(t, w2_ref[...], preferred_element_type=jnp.float32) + b2_ref[...]
    t = t * (g_ref[...] * _BN_SCALE) + bt_ref[...]
    ho = jnp.maximum(t, 0.0)
    ho_ref[...] = ho

    onehot = (batch_ref[...] ==
              lax.broadcasted_iota(jnp.int32, (BN, G), 1)).astype(jnp.float32)
    s_blk = lax.dot_general(onehot, ho, (((0,), (0,)), ((), ())),
                            preferred_element_type=jnp.float32)
    c_blk = lax.dot_general(onehot, jnp.ones((BN, H), jnp.float32),
                            (((0,), (0,)), ((), ())),
                            preferred_element_type=jnp.float32)

    @pl.when(i == 0)
    def _():
        s_ref[...] = jnp.zeros_like(s_ref)
        cnt_ref[...] = jnp.zeros_like(cnt_ref)

    s_ref[...] += s_blk
    cnt_ref[...] += c_blk

    @pl.when(i == pl.num_programs(0) - 1)
    def _():
        emb_ref[...] = s_ref[...] / jnp.maximum(cnt_ref[...], 1.0)


def _addvn_body(h_ref, batch_ref, s_ref, vnp_ref, w1_ref, b1_ref, w2_ref,
                b2_ref, ho_ref, vn_ref):
    u = jnp.maximum(
        jnp.dot(s_ref[...], w1_ref[...], preferred_element_type=jnp.float32)
        + b1_ref[...], 0.0)
    u = jnp.dot(u, w2_ref[...], preferred_element_type=jnp.float32) + b2_ref[...]
    vn_new = vnp_ref[...] + u
    vn_ref[...] = vn_new
    onehot = (batch_ref[...] ==
              lax.broadcasted_iota(jnp.int32, (BN, G), 1)).astype(jnp.float32)
    ho_ref[...] = h_ref[...] + jnp.dot(
        onehot, vn_new, preferred_element_type=jnp.float32)


def _row_spec(blk):
    return pl.BlockSpec((blk, H), lambda i: (i, 0))


def _full_spec(r):
    return pl.BlockSpec((r, H), lambda i: (0, 0))


_proj = pl.pallas_call(
    _proj_body,
    grid=(N // BN,),
    in_specs=[_row_spec(BN), _full_spec(H), pl.BlockSpec((1, H), lambda i: (0, 0))],
    out_specs=_row_spec(BN),
    out_shape=jax.ShapeDtypeStruct((N, H), jnp.float32),
)

_edge_feat = pl.pallas_call(
    _edge_feat_body,
    grid=(E // BE,),
    in_specs=[pl.BlockSpec((BE, 4), lambda i: (i, 0)),
              pl.BlockSpec((4, H), lambda i: (0, 0)),
              pl.BlockSpec((1, H), lambda i: (0, 0))],
    out_specs=_row_spec(BE),
    out_shape=jax.ShapeDtypeStruct((E, H), jnp.float32),
)

_layer = pl.pallas_call(
    _layer_body,
    grid=(N // BN,),
    in_specs=[_row_spec(BN), _row_spec(BN), _row_spec(BN),
              pl.BlockSpec((BN, 1), lambda i: (i, 0)),
              _full_spec(H), pl.BlockSpec((1, H), lambda i: (0, 0)),
              _full_spec(H), pl.BlockSpec((1, H), lambda i: (0, 0)),
              pl.BlockSpec((1, H), lambda i: (0, 0)),
              pl.BlockSpec((1, H), lambda i: (0, 0))],
    out_specs=[_row_spec(BN), _full_spec(G), _full_spec(G), _full_spec(G)],
    out_shape=[jax.ShapeDtypeStruct((N, H), jnp.float32),
               jax.ShapeDtypeStruct((G, H), jnp.float32),
               jax.ShapeDtypeStruct((G, H), jnp.float32),
               jax.ShapeDtypeStruct((G, H), jnp.float32)],
)

_addvn = pl.pallas_call(
    _addvn_body,
    grid=(N // BN,),
    in_specs=[_row_spec(BN),
              pl.BlockSpec((BN, 1), lambda i: (i, 0)),
              _full_spec(G), _full_spec(G),
              _full_spec(H), pl.BlockSpec((1, H), lambda i: (0, 0)),
              _full_spec(H), pl.BlockSpec((1, H), lambda i: (0, 0))],
    out_specs=[_row_spec(BN), _full_spec(G)],
    out_shape=[jax.ShapeDtypeStruct((N, H), jnp.float32),
               jax.ShapeDtypeStruct((G, H), jnp.float32)],
)


def kernel(x, edge_index, edge_attr, batch, proj_W, proj_b, edge_W, edge_b,
           conv_W1, conv_b1, conv_W2, conv_b2, bn_gamma, bn_beta,
           vn_W1, vn_b1, vn_W2, vn_b2):
    src = edge_index[0]
    dst = edge_index[1]
    batch2 = batch.reshape(N, 1)

    proj_b2 = proj_b.reshape(1, H)
    edge_b2 = edge_b.reshape(1, H)
    vn_b1_2 = vn_b1.reshape(1, H)
    vn_b2_2 = vn_b2.reshape(1, H)

    h = _proj(x, proj_W, proj_b2)
    e = _edge_feat(edge_attr, edge_W, edge_b2)

    vn = jnp.zeros((G, H), jnp.float32)
    s_prev = None
    emb = None
    for i in range(3):
        if i > 0:
            h, vn = _addvn(h, batch2, s_prev, vn, vn_W1, vn_b1_2,
                           vn_W2, vn_b2_2)
        agg = _sc_aggr(h, e, src, dst)
        h, s_prev, _cnt, emb = _layer(
            h, agg[:N], agg[NPAD:NPAD + N], batch2,
            conv_W1[i], conv_b1[i].reshape(1, H),
            conv_W2[i], conv_b2[i].reshape(1, H),
            bn_gamma[i].reshape(1, H), bn_beta[i].reshape(1, H))
    return (h, emb)


# speedup vs baseline: 6.3041x; 1.1795x over previous
import functools

import jax
import jax.numpy as jnp
import numpy as np
from jax import lax
from jax.experimental import pallas as pl
from jax.experimental.pallas import tpu as pltpu
from jax.experimental.pallas import tpu_sc as plsc

N = 10000
E = 320000
H = 128
G = 64

NC = 2
NS = 16
EPW = E // (NC * NS)
C = 80
NCHUNK = EPW // C
NPAD = 10240
RPT = NPAD // NS
ZR = 128

BN = 2000
BE = 6400

_BN_SCALE = np.float32(1.0 / np.sqrt(1.0 + 1e-5))


def _sc_aggr_body(h_hbm, e_hbm, src_hbm, dst_hbm, out_hbm,
                  srcA, dstA, srcB, dstB, rowsA, rowsB, eA, eB, accum,
                  isemA, isemB, esemA, esemB, gsemA, gsemB, ssemA, ssemB):
    c = lax.axis_index("c")
    s = lax.axis_index("s")
    wbase = c * (NS * EPW) + s * EPW

    banks = ((srcA, dstA, rowsA, eA, isemA, esemA, gsemA, ssemA),
             (srcB, dstB, rowsB, eB, isemB, esemB, gsemB, ssemB))


    def scat_start(bank):
        pltpu.async_copy(bank[2], accum.at[bank[1]], bank[7], add=True)

    def scat_wait(bank):
        pltpu.make_async_copy(bank[2], accum.at[bank[1]], bank[7]).wait()

    def idx_copies(j, bank):
        base = wbase + j * C
        return (pltpu.make_async_copy(src_hbm.at[pl.ds(base, C)], bank[0],
                                      bank[4]),
                pltpu.make_async_copy(dst_hbm.at[pl.ds(base, C)], bank[1],
                                      bank[4]))

    def e_copy(j, bank):
        base = wbase + j * C
        return pltpu.make_async_copy(e_hbm.at[pl.ds(base, C)], bank[3],
                                     bank[5])

    def g_copy(bank):
        return pltpu.make_async_copy(h_hbm.at[bank[0]], bank[2], bank[6])

    for cp in idx_copies(0, banks[0]):
        cp.start()
    for cp in idx_copies(1, banks[1]):
        cp.start()
    for cp in idx_copies(0, banks[0]):
        cp.wait()
    g_copy(banks[0]).start()
    e_copy(0, banks[0]).start()

    zero16 = jnp.zeros((16,), jnp.float32)

    @plsc.parallel_loop(0, C, unroll=2)
    def _(i):
        for r in range(8):
            rowsB[i, pl.ds(r * 16, 16)] = zero16

    for k in range(RPT // C):
        pltpu.sync_copy(rowsB, accum.at[pl.ds(s * RPT + k * C, C)])
    plsc.subcore_barrier()

    pltpu.async_copy(rowsB, accum.at[dstA], ssemB, add=True)

    def compute(bank):
        rows, ebuf = bank[2], bank[3]

        @plsc.parallel_loop(0, C, unroll=8)
        def _(i):
            for r in range(8):
                sl = pl.ds(r * 16, 16)
                rows[i, sl] = jnp.maximum(rows[i, sl] + ebuf[i, sl], 0.0)

    def phase(jcur, cur, nxt):
        jnext = jcur + 1
        g_copy(cur).wait()
        for cp in idx_copies(jnext, nxt):
            cp.wait()
        scat_wait(nxt)
        g_copy(nxt).start()
        e_copy(jnext, nxt).start()
        e_copy(jcur, cur).wait()
        compute(cur)
        scat_start(cur)

        @pl.when(jcur + 2 < NCHUNK)
        def _():
            for cp in idx_copies(jcur + 2, cur):
                cp.start()

    def pair(j2, carry):
        phase(2 * j2, banks[0], banks[1])
        phase(2 * j2 + 1, banks[1], banks[0])
        return carry

    lax.fori_loop(0, (NCHUNK - 1) // 2, pair, 0)

    g_copy(banks[0]).wait()
    e_copy(NCHUNK - 1, banks[0]).wait()
    compute(banks[0])
    scat_start(banks[0])
    scat_wait(banks[1])
    scat_wait(banks[0])

    plsc.subcore_barrier()

    nchunks_out = RPT // C

    def out_cp(k, bank):
        off = s * RPT + k * C
        return pltpu.make_async_copy(bank[2],
                                     out_hbm.at[pl.ds(c * NPAD + off, C)],
                                     bank[6])

    for k in range(nchunks_out):
        bank = banks[k % 2]
        if k >= 2:
            out_cp(k - 2, bank).wait()
        pltpu.sync_copy(accum.at[pl.ds(s * RPT + k * C, C)], bank[2])
        out_cp(k, bank).start()
    out_cp(nchunks_out - 2, banks[nchunks_out % 2]).wait()
    out_cp(nchunks_out - 1, banks[(nchunks_out - 1) % 2]).wait()


@functools.cache
def _sc_aggr_kernel():
    return pl.kernel(
        _sc_aggr_body,
        out_type=jax.ShapeDtypeStruct((NC * NPAD, H), jnp.float32),
        mesh=plsc.VectorSubcoreMesh(core_axis_name="c", subcore_axis_name="s",
                                    num_cores=NC, num_subcores=NS),
        scratch_types=[
            pltpu.VMEM((C,), jnp.int32),
            pltpu.VMEM((C,), jnp.int32),
            pltpu.VMEM((C,), jnp.int32),
            pltpu.VMEM((C,), jnp.int32),
            pltpu.VMEM((C, H), jnp.float32),
            pltpu.VMEM((C, H), jnp.float32),
            pltpu.VMEM((C, H), jnp.float32),
            pltpu.VMEM((C, H), jnp.float32),
            pltpu.VMEM_SHARED((NPAD, H), jnp.float32),
            pltpu.SemaphoreType.DMA,
            pltpu.SemaphoreType.DMA,
            pltpu.SemaphoreType.DMA,
            pltpu.SemaphoreType.DMA,
            pltpu.SemaphoreType.DMA,
            pltpu.SemaphoreType.DMA,
            pltpu.SemaphoreType.DMA,
            pltpu.SemaphoreType.DMA,
        ],
    )


def _sc_aggr(h, e, src, dst):
    return _sc_aggr_kernel()(h, e, src, dst)


def _proj_body(x_ref, w_ref, b_ref, o_ref):
    o_ref[...] = (
        jnp.dot(x_ref[...], w_ref[...], preferred_element_type=jnp.float32)
        + b_ref[...]
    )


def _edge_feat_body(at_ref, w_ref, b_ref, o_ref):
    o_ref[...] = lax.dot_general(
        at_ref[...], w_ref[...], (((0,), (0,)), ((), ())),
        preferred_element_type=jnp.float32) + b_ref[...]


def _layer_body(h_ref, a0_ref, a1_ref, batch_ref, w1_ref, b1_ref, w2_ref,
                b2_ref, g_ref, bt_ref, ho_ref, s_ref, cnt_ref, emb_ref):
    i = pl.program_id(0)
    z = h_ref[...] + a0_ref[...] + a1_ref[...]
    t = jnp.maximum(
        jnp.dot(z, w1_ref[...], preferred_element_type=jnp.float32)
        + b1_ref[...], 0.0)
    t = jnp.dot(t, w2_ref[...], preferred_element_type=jnp.float32) + b2_ref[...]
    t = t * (g_ref[...] * _BN_SCALE) + bt_ref[...]
    ho = jnp.maximum(t, 0.0)
    ho_ref[...] = ho

    onehot = (batch_ref[...] ==
              lax.broadcasted_iota(jnp.int32, (BN, G), 1)).astype(jnp.float32)
    s_blk = lax.dot_general(onehot, ho, (((0,), (0,)), ((), ())),
                            preferred_element_type=jnp.float32)
    c_blk = lax.dot_general(onehot, jnp.ones((BN, H), jnp.float32),
                            (((0,), (0,)), ((), ())),
                            preferred_element_type=jnp.float32)

    @pl.when(i == 0)
    def _():
        s_ref[...] = jnp.zeros_like(s_ref)
        cnt_ref[...] = jnp.zeros_like(cnt_ref)

    s_ref[...] += s_blk
    cnt_ref[...] += c_blk

    @pl.when(i == pl.num_programs(0) - 1)
    def _():
        emb_ref[...] = s_ref[...] / jnp.maximum(cnt_ref[...], 1.0)


def _addvn_body(h_ref, batch_ref, s_ref, vnp_ref, w1_ref, b1_ref, w2_ref,
                b2_ref, ho_ref, vn_ref):
    u = jnp.maximum(
        jnp.dot(s_ref[...], w1_ref[...], preferred_element_type=jnp.float32)
        + b1_ref[...], 0.0)
    u = jnp.dot(u, w2_ref[...], preferred_element_type=jnp.float32) + b2_ref[...]
    vn_new = vnp_ref[...] + u
    vn_ref[...] = vn_new
    onehot = (batch_ref[...] ==
              lax.broadcasted_iota(jnp.int32, (BN, G), 1)).astype(jnp.float32)
    ho_ref[...] = h_ref[...] + jnp.dot(
        onehot, vn_new, preferred_element_type=jnp.float32)


def _row_spec(blk):
    return pl.BlockSpec((blk, H), lambda i: (i, 0))


def _full_spec(r):
    return pl.BlockSpec((r, H), lambda i: (0, 0))


_proj = pl.pallas_call(
    _proj_body,
    grid=(N // BN,),
    in_specs=[_row_spec(BN), _full_spec(H), pl.BlockSpec((1, H), lambda i: (0, 0))],
    out_specs=_row_spec(BN),
    out_shape=jax.ShapeDtypeStruct((N, H), jnp.float32),
)

_edge_feat = pl.pallas_call(
    _edge_feat_body,
    grid=(E // BE,),
    in_specs=[pl.BlockSpec((4, BE), lambda i: (0, i)),
              pl.BlockSpec((4, H), lambda i: (0, 0)),
              pl.BlockSpec((1, H), lambda i: (0, 0))],
    out_specs=_row_spec(BE),
    out_shape=jax.ShapeDtypeStruct((E, H), jnp.float32),
)

_layer = pl.pallas_call(
    _layer_body,
    grid=(N // BN,),
    in_specs=[_row_spec(BN), _row_spec(BN), _row_spec(BN),
              pl.BlockSpec((BN, 1), lambda i: (i, 0)),
              _full_spec(H), pl.BlockSpec((1, H), lambda i: (0, 0)),
              _full_spec(H), pl.BlockSpec((1, H), lambda i: (0, 0)),
              pl.BlockSpec((1, H), lambda i: (0, 0)),
              pl.BlockSpec((1, H), lambda i: (0, 0))],
    out_specs=[_row_spec(BN), _full_spec(G), _full_spec(G), _full_spec(G)],
    out_shape=[jax.ShapeDtypeStruct((N, H), jnp.float32),
               jax.ShapeDtypeStruct((G, H), jnp.float32),
               jax.ShapeDtypeStruct((G, H), jnp.float32),
               jax.ShapeDtypeStruct((G, H), jnp.float32)],
)

_addvn = pl.pallas_call(
    _addvn_body,
    grid=(N // BN,),
    in_specs=[_row_spec(BN),
              pl.BlockSpec((BN, 1), lambda i: (i, 0)),
              _full_spec(G), _full_spec(G),
              _full_spec(H), pl.BlockSpec((1, H), lambda i: (0, 0)),
              _full_spec(H), pl.BlockSpec((1, H), lambda i: (0, 0))],
    out_specs=[_row_spec(BN), _full_spec(G)],
    out_shape=[jax.ShapeDtypeStruct((N, H), jnp.float32),
               jax.ShapeDtypeStruct((G, H), jnp.float32)],
)


def kernel(x, edge_index, edge_attr, batch, proj_W, proj_b, edge_W, edge_b,
           conv_W1, conv_b1, conv_W2, conv_b2, bn_gamma, bn_beta,
           vn_W1, vn_b1, vn_W2, vn_b2):
    src = edge_index[0]
    dst = edge_index[1]
    batch2 = batch.reshape(N, 1)

    proj_b2 = proj_b.reshape(1, H)
    edge_b2 = edge_b.reshape(1, H)
    vn_b1_2 = vn_b1.reshape(1, H)
    vn_b2_2 = vn_b2.reshape(1, H)

    h = _proj(x, proj_W, proj_b2)
    e = _edge_feat(edge_attr.T, edge_W, edge_b2)

    vn = jnp.zeros((G, H), jnp.float32)
    s_prev = None
    emb = None
    for i in range(3):
        if i > 0:
            h, vn = _addvn(h, batch2, s_prev, vn, vn_W1, vn_b1_2,
                           vn_W2, vn_b2_2)
        agg = _sc_aggr(h, e, src, dst)
        h, s_prev, _cnt, emb = _layer(
            h, agg[:N], agg[NPAD:NPAD + N], batch2,
            conv_W1[i], conv_b1[i].reshape(1, H),
            conv_W2[i], conv_b2[i].reshape(1, H),
            bn_gamma[i].reshape(1, H), bn_beta[i].reshape(1, H))
    return (h, emb)


# speedup vs baseline: 6.5757x; 1.0431x over previous
import functools

import jax
import jax.numpy as jnp
import numpy as np
from jax import lax
from jax.experimental import pallas as pl
from jax.experimental.pallas import tpu as pltpu
from jax.experimental.pallas import tpu_sc as plsc

N = 10000
E = 320000
H = 128
G = 64

NC = 2
NS = 16
EPW = E // (NC * NS)
C = 80
NCHUNK = EPW // C
NPAD = 10240
RPT = NPAD // NS
ZR = 128

BN = 2000
BE = 6400

_BN_SCALE = np.float32(1.0 / np.sqrt(1.0 + 1e-5))


def _sc_aggr_body(h_hbm, e_hbm, src_hbm, dst_hbm, out_hbm,
                  srcA, dstA, srcB, dstB, rowsA, rowsB, eA, eB, accum,
                  isemA, isemB, esemA, esemB, gsemA, gsemB, ssemA, ssemB):
    c = lax.axis_index("c")
    s = lax.axis_index("s")
    wbase = c * (NS * EPW) + s * EPW

    banks = ((srcA, dstA, rowsA, eA, isemA, esemA, gsemA, ssemA),
             (srcB, dstB, rowsB, eB, isemB, esemB, gsemB, ssemB))


    def scat_start(bank):
        pltpu.async_copy(bank[2], accum.at[bank[1]], bank[7], add=True)

    def scat_wait(bank):
        pltpu.make_async_copy(bank[2], accum.at[bank[1]], bank[7]).wait()

    def idx_copies(j, bank):
        base = wbase + j * C
        return (pltpu.make_async_copy(src_hbm.at[pl.ds(base, C)], bank[0],
                                      bank[4]),
                pltpu.make_async_copy(dst_hbm.at[pl.ds(base, C)], bank[1],
                                      bank[4]))

    def e_copy(j, bank):
        base = wbase + j * C
        return pltpu.make_async_copy(e_hbm.at[pl.ds(base, C)], bank[3],
                                     bank[5])

    def g_copy(bank):
        return pltpu.make_async_copy(h_hbm.at[bank[0]], bank[2], bank[6])

    for cp in idx_copies(0, banks[0]):
        cp.start()
    for cp in idx_copies(1, banks[1]):
        cp.start()
    for cp in idx_copies(0, banks[0]):
        cp.wait()
    g_copy(banks[0]).start()
    e_copy(0, banks[0]).start()

    zero16 = jnp.zeros((16,), jnp.float32)

    @plsc.parallel_loop(0, C, unroll=2)
    def _(i):
        for r in range(8):
            rowsB[i, pl.ds(r * 16, 16)] = zero16

    for k in range(RPT // C):
        pltpu.sync_copy(rowsB, accum.at[pl.ds(s * RPT + k * C, C)])
    plsc.subcore_barrier()

    pltpu.async_copy(rowsB, accum.at[dstA], ssemB, add=True)

    def compute(bank):
        rows, ebuf = bank[2], bank[3]

        @plsc.parallel_loop(0, C, unroll=4)
        def _(i):
            for r in range(8):
                sl = pl.ds(r * 16, 16)
                rows[i, sl] = jnp.maximum(rows[i, sl] + ebuf[i, sl], 0.0)

    def phase(jcur, cur, nxt):
        jnext = jcur + 1
        g_copy(cur).wait()
        for cp in idx_copies(jnext, nxt):
            cp.wait()
        scat_wait(nxt)
        g_copy(nxt).start()
        e_copy(jnext, nxt).start()
        e_copy(jcur, cur).wait()
        compute(cur)
        scat_start(cur)

        @pl.when(jcur + 2 < NCHUNK)
        def _():
            for cp in idx_copies(jcur + 2, cur):
                cp.start()

    def pair(j2, carry):
        phase(2 * j2, banks[0], banks[1])
        phase(2 * j2 + 1, banks[1], banks[0])
        return carry

    lax.fori_loop(0, (NCHUNK - 1) // 2, pair, 0)

    g_copy(banks[0]).wait()
    e_copy(NCHUNK - 1, banks[0]).wait()
    compute(banks[0])
    scat_start(banks[0])
    scat_wait(banks[1])
    scat_wait(banks[0])

    plsc.subcore_barrier()

    nchunks_out = RPT // C

    def out_cp(k, bank):
        off = s * RPT + k * C
        return pltpu.make_async_copy(bank[2],
                                     out_hbm.at[pl.ds(c * NPAD + off, C)],
                                     bank[6])

    for k in range(nchunks_out):
        bank = banks[k % 2]
        if k >= 2:
            out_cp(k - 2, bank).wait()
        pltpu.sync_copy(accum.at[pl.ds(s * RPT + k * C, C)], bank[2])
        out_cp(k, bank).start()
    out_cp(nchunks_out - 2, banks[nchunks_out % 2]).wait()
    out_cp(nchunks_out - 1, banks[(nchunks_out - 1) % 2]).wait()


@functools.cache
def _sc_aggr_kernel():
    return pl.kernel(
        _sc_aggr_body,
        out_type=jax.ShapeDtypeStruct((NC * NPAD, H), jnp.float32),
        mesh=plsc.VectorSubcoreMesh(core_axis_name="c", subcore_axis_name="s",
                                    num_cores=NC, num_subcores=NS),
        scratch_types=[
            pltpu.VMEM((C,), jnp.int32),
            pltpu.VMEM((C,), jnp.int32),
            pltpu.VMEM((C,), jnp.int32),
            pltpu.VMEM((C,), jnp.int32),
            pltpu.VMEM((C, H), jnp.float32),
            pltpu.VMEM((C, H), jnp.float32),
            pltpu.VMEM((C, H), jnp.float32),
            pltpu.VMEM((C, H), jnp.float32),
            pltpu.VMEM_SHARED((NPAD, H), jnp.float32),
            pltpu.SemaphoreType.DMA,
            pltpu.SemaphoreType.DMA,
            pltpu.SemaphoreType.DMA,
            pltpu.SemaphoreType.DMA,
            pltpu.SemaphoreType.DMA,
            pltpu.SemaphoreType.DMA,
            pltpu.SemaphoreType.DMA,
            pltpu.SemaphoreType.DMA,
        ],
    )


def _sc_aggr(h, e, src, dst):
    return _sc_aggr_kernel()(h, e, src, dst)


def _proj_body(x_ref, w_ref, b_ref, o_ref):
    o_ref[...] = (
        jnp.dot(x_ref[...], w_ref[...], preferred_element_type=jnp.float32)
        + b_ref[...]
    )


def _edge_feat_body(at_ref, w_ref, b_ref, o_ref):
    o_ref[...] = lax.dot_general(
        at_ref[...], w_ref[...], (((0,), (0,)), ((), ())),
        preferred_element_type=jnp.float32) + b_ref[...]


def _layer_body(h_ref, a0_ref, a1_ref, batch_ref, w1_ref, b1_ref, w2_ref,
                b2_ref, g_ref, bt_ref, ho_ref, s_ref, cnt_ref, emb_ref):
    i = pl.program_id(0)
    z = h_ref[...] + a0_ref[...] + a1_ref[...]
    t = jnp.maximum(
        jnp.dot(z, w1_ref[...], preferred_element_type=jnp.float32)
        + b1_ref[...], 0.0)
    t = jnp.dot(t, w2_ref[...], preferred_element_type=jnp.float32) + b2_ref[...]
    t = t * (g_ref[...] * _BN_SCALE) + bt_ref[...]
    ho = jnp.maximum(t, 0.0)
    ho_ref[...] = ho

    onehot = (batch_ref[...] ==
              lax.broadcasted_iota(jnp.int32, (BN, G), 1)).astype(jnp.float32)
    s_blk = lax.dot_general(onehot, ho, (((0,), (0,)), ((), ())),
                            preferred_element_type=jnp.float32)
    c_blk = lax.dot_general(onehot, jnp.ones((BN, H), jnp.float32),
                            (((0,), (0,)), ((), ())),
                            preferred_element_type=jnp.float32)

    @pl.when(i == 0)
    def _():
        s_ref[...] = jnp.zeros_like(s_ref)
        cnt_ref[...] = jnp.zeros_like(cnt_ref)

    s_ref[...] += s_blk
    cnt_ref[...] += c_blk

    @pl.when(i == pl.num_programs(0) - 1)
    def _():
        emb_ref[...] = s_ref[...] / jnp.maximum(cnt_ref[...], 1.0)


def _addvn_body(h_ref, batch_ref, s_ref, vnp_ref, w1_ref, b1_ref, w2_ref,
                b2_ref, ho_ref, vn_ref):
    u = jnp.maximum(
        jnp.dot(s_ref[...], w1_ref[...], preferred_element_type=jnp.float32)
        + b1_ref[...], 0.0)
    u = jnp.dot(u, w2_ref[...], preferred_element_type=jnp.float32) + b2_ref[...]
    vn_new = vnp_ref[...] + u
    vn_ref[...] = vn_new
    onehot = (batch_ref[...] ==
              lax.broadcasted_iota(jnp.int32, (BN, G), 1)).astype(jnp.float32)
    ho_ref[...] = h_ref[...] + jnp.dot(
        onehot, vn_new, preferred_element_type=jnp.float32)


def _row_spec(blk):
    return pl.BlockSpec((blk, H), lambda i: (i, 0))


def _full_spec(r):
    return pl.BlockSpec((r, H), lambda i: (0, 0))


_proj = pl.pallas_call(
    _proj_body,
    grid=(N // BN,),
    in_specs=[_row_spec(BN), _full_spec(H), pl.BlockSpec((1, H), lambda i: (0, 0))],
    out_specs=_row_spec(BN),
    out_shape=jax.ShapeDtypeStruct((N, H), jnp.float32),
)

_edge_feat = pl.pallas_call(
    _edge_feat_body,
    grid=(E // BE,),
    in_specs=[pl.BlockSpec((4, BE), lambda i: (0, i)),
              pl.BlockSpec((4, H), lambda i: (0, 0)),
              pl.BlockSpec((1, H), lambda i: (0, 0))],
    out_specs=_row_spec(BE),
    out_shape=jax.ShapeDtypeStruct((E, H), jnp.float32),
)

_layer = pl.pallas_call(
    _layer_body,
    grid=(N // BN,),
    in_specs=[_row_spec(BN), _row_spec(BN), _row_spec(BN),
              pl.BlockSpec((BN, 1), lambda i: (i, 0)),
              _full_spec(H), pl.BlockSpec((1, H), lambda i: (0, 0)),
              _full_spec(H), pl.BlockSpec((1, H), lambda i: (0, 0)),
              pl.BlockSpec((1, H), lambda i: (0, 0)),
              pl.BlockSpec((1, H), lambda i: (0, 0))],
    out_specs=[_row_spec(BN), _full_spec(G), _full_spec(G), _full_spec(G)],
    out_shape=[jax.ShapeDtypeStruct((N, H), jnp.float32),
               jax.ShapeDtypeStruct((G, H), jnp.float32),
               jax.ShapeDtypeStruct((G, H), jnp.float32),
               jax.ShapeDtypeStruct((G, H), jnp.float32)],
)

_addvn = pl.pallas_call(
    _addvn_body,
    grid=(N // BN,),
    in_specs=[_row_spec(BN),
              pl.BlockSpec((BN, 1), lambda i: (i, 0)),
              _full_spec(G), _full_spec(G),
              _full_spec(H), pl.BlockSpec((1, H), lambda i: (0, 0)),
              _full_spec(H), pl.BlockSpec((1, H), lambda i: (0, 0))],
    out_specs=[_row_spec(BN), _full_spec(G)],
    out_shape=[jax.ShapeDtypeStruct((N, H), jnp.float32),
               jax.ShapeDtypeStruct((G, H), jnp.float32)],
)


def kernel(x, edge_index, edge_attr, batch, proj_W, proj_b, edge_W, edge_b,
           conv_W1, conv_b1, conv_W2, conv_b2, bn_gamma, bn_beta,
           vn_W1, vn_b1, vn_W2, vn_b2):
    src = edge_index[0]
    dst = edge_index[1]
    batch2 = batch.reshape(N, 1)

    proj_b2 = proj_b.reshape(1, H)
    edge_b2 = edge_b.reshape(1, H)
    vn_b1_2 = vn_b1.reshape(1, H)
    vn_b2_2 = vn_b2.reshape(1, H)

    h = _proj(x, proj_W, proj_b2)
    e = _edge_feat(edge_attr.T, edge_W, edge_b2)

    vn = jnp.zeros((G, H), jnp.float32)
    s_prev = None
    emb = None
    for i in range(3):
        if i > 0:
            h, vn = _addvn(h, batch2, s_prev, vn, vn_W1, vn_b1_2,
                           vn_W2, vn_b2_2)
        agg = _sc_aggr(h, e, src, dst)
        h, s_prev, _cnt, emb = _layer(
            h, agg[:N], agg[NPAD:NPAD + N], batch2,
            conv_W1[i], conv_b1[i].reshape(1, H),
            conv_W2[i], conv_b2[i].reshape(1, H),
            bn_gamma[i].reshape(1, H), bn_beta[i].reshape(1, H))
    return (h, emb)
